# baseline (speedup 1.0000x reference)
import functools

import jax
import jax.numpy as jnp
from jax import lax
from jax.experimental import pallas as pl
from jax.experimental.pallas import tpu as pltpu
from jax.experimental.pallas import tpu_sc as plsc

GENOME = 100000
H1 = 128
H2 = 512
BATCH = 16384
NIDX = 327680

NC = 2
NS = 16
NW = NC * NS

ROWS_PER_TILE = BATCH // NW
GROWS = 128
TAIL0 = BATCH
TAIL_N = NIDX - TAIL0
PAIRS_PER_TILE = TAIL_N // NW
PCHUNK = 2432
HROWS = 784
GPAD = HROWS * H1
HCHUNK = 112
GCH = 25088

_SC_MESH = plsc.VectorSubcoreMesh(core_axis_name="c", subcore_axis_name="s")
_SC_PARAMS = pltpu.CompilerParams(needs_layout_passes=False)


def _hist_body(idx_hbm, w_hbm, hist_hbm, hist_v, pi_v, pw_v, row_idx_v, shared):
    cid = lax.axis_index("c")
    sid = lax.axis_index("s")
    wid = sid * NC + cid
    zero16 = jnp.zeros((16,), jnp.float32)
    lane = lax.iota(jnp.int32, 16)

    def _zero(i, carry):
        for u in range(8):
            hist_v[i, pl.ds(u * 16, 16)] = zero16
        return carry

    lax.fori_loop(0, HROWS, _zero, 0)

    for j in range(HROWS // HCHUNK):
        for u in range(HCHUNK // 16):
            row_idx_v[j, pl.ds(u * 16, 16)] = j * HCHUNK + u * 16 + lane

    pbase = TAIL0 + wid * PAIRS_PER_TILE
    for c in range(PAIRS_PER_TILE // PCHUNK):
        pltpu.sync_copy(idx_hbm.at[pl.ds(pbase + c * PCHUNK, PCHUNK)], pi_v)
        pltpu.sync_copy(w_hbm.at[pl.ds(pbase + c * PCHUNK, PCHUNK)], pw_v)

        def _scat(v, carry):
            ii = pi_v[pl.ds(v * 16, 16)]
            ww = pw_v[pl.ds(v * 16, 16)]
            ks, ws = plsc.sort_key_val(ii, ww)
            s = plsc.cumsum(ws)
            cnt, last = plsc.scan_count(ks)
            first = jnp.logical_and(cnt == 1, lane > 0)
            kr = lax.shift_right_logical(ks, 7)
            kc = jnp.bitwise_and(ks, 127)
            plsc.addupdate_scatter(hist_v, [kr, kc], s, mask=last)
            plsc.addupdate_scatter(hist_v, [kr, kc], ws - s, mask=first)
            return carry

        lax.fori_loop(0, PCHUNK // 16, _scat, 0)

    @pl.when(sid == 0)
    def _():
        pltpu.sync_copy(hist_v, shared)

    plsc.subcore_barrier()

    @pl.when(sid != 0)
    def _():
        for j in range(HROWS // HCHUNK):
            pltpu.sync_copy(hist_v.at[pl.ds(j * HCHUNK, HCHUNK)],
                            shared.at[row_idx_v.at[j]], add=True)

    plsc.subcore_barrier()

    @pl.when(sid == 0)
    def _():
        pltpu.sync_copy(shared, hist_hbm.at[cid])


_sc_hist = functools.partial(
    pl.kernel,
    out_type=jax.ShapeDtypeStruct((NC, HROWS, H1), jnp.float32),
    mesh=_SC_MESH,
    compiler_params=_SC_PARAMS,
    scratch_types=[
        pltpu.VMEM((HROWS, H1), jnp.float32),
        pltpu.VMEM((PCHUNK,), jnp.int32),
        pltpu.VMEM((PCHUNK,), jnp.float32),
        pltpu.VMEM((HROWS // HCHUNK, HCHUNK), jnp.int32),
        pltpu.VMEM_SHARED((HROWS, H1), jnp.float32),
    ],
)(_hist_body)


def _gath_body(idx_hbm, tab_hbm, gath_hbm, idx_v, rows0_v, rows1_v, sem0, sem1):
    wid = lax.axis_index("s") * NC + lax.axis_index("c")
    base = wid * ROWS_PER_TILE
    pltpu.sync_copy(idx_hbm.at[pl.ds(base, ROWS_PER_TILE)], idx_v)
    bufs = (rows0_v, rows1_v)
    sems = (sem0, sem1)
    nch = ROWS_PER_TILE // GROWS
    copies = []
    for c in range(nch):
        b = c % 2
        copies.append(pltpu.async_copy(
            tab_hbm.at[idx_v.at[pl.ds(c * GROWS, GROWS)]], bufs[b], sems[b]))
        if c >= 1:
            copies[c - 1].wait()
            pltpu.sync_copy(bufs[1 - b],
                            gath_hbm.at[pl.ds(base + (c - 1) * GROWS, GROWS)])
    copies[nch - 1].wait()
    pltpu.sync_copy(bufs[(nch - 1) % 2],
                    gath_hbm.at[pl.ds(base + (nch - 1) * GROWS, GROWS)])


_sc_gather = functools.partial(
    pl.kernel,
    out_type=jax.ShapeDtypeStruct((BATCH, H1), jnp.float32),
    mesh=_SC_MESH,
    compiler_params=_SC_PARAMS,
    scratch_types=[
        pltpu.VMEM((ROWS_PER_TILE,), jnp.int32),
        pltpu.VMEM((GROWS, H1), jnp.float32),
        pltpu.VMEM((GROWS, H1), jnp.float32),
        pltpu.SemaphoreType.DMA,
        pltpu.SemaphoreType.DMA,
    ],
)(_gath_body)


def _leaky(v):
    return jnp.where(v >= 0, v, 0.01 * v)


def _mv_body(acc_ref, tab_ref, out_ref):
    j = pl.program_id(0)
    acc = jnp.sum(acc_ref[...], axis=0, keepdims=True)
    lids = j * GCH + lax.broadcasted_iota(jnp.int32, (1, GCH), 1)
    acc = jnp.where(lids < GENOME, acc, 0.0)
    rids = j * GCH + lax.broadcasted_iota(jnp.int32, (GCH, 1), 0)
    tab = jnp.where(rids < GENOME, tab_ref[...], 0.0)
    part = lax.dot_general(acc, tab, (((1,), (0,)), ((), ())),
                           precision=lax.Precision.HIGHEST,
                           preferred_element_type=jnp.float32)

    @pl.when(j == 0)
    def _():
        out_ref[...] = jnp.zeros_like(out_ref)

    out_ref[...] += part


def _tail_matvec(hist, table):
    return pl.pallas_call(
        _mv_body,
        grid=(GPAD // GCH,),
        in_specs=[
            pl.BlockSpec((NC, GCH), lambda j: (0, j)),
            pl.BlockSpec((GCH, H1), lambda j: (j, 0)),
        ],
        out_specs=pl.BlockSpec((1, H1), lambda j: (0, 0)),
        out_shape=jax.ShapeDtypeStruct((1, H1), jnp.float32),
    )(hist, table)


RBLK = 2048


def _mlp_body(gath_ref, w_ref, tail_ref, W2_ref, b2_ref, W3_ref, b3_ref,
              g_ref, out_ref):
    i = pl.program_id(0)
    x = gath_ref[...] * w_ref[...]
    rid = i * RBLK + lax.broadcasted_iota(jnp.int32, (RBLK, 1), 0)
    is_last = jnp.where(rid == BATCH - 1, 1.0, 0.0)
    x = x + is_last * tail_ref[...]
    x = _leaky(x)
    h = lax.dot_general(x, W2_ref[...], (((1,), (1,)), ((), ())),
                        preferred_element_type=jnp.float32) + b2_ref[...]
    h = _leaky(h)
    y = lax.dot_general(h, W3_ref[...], (((1,), (1,)), ((), ())),
                        preferred_element_type=jnp.float32) + b3_ref[...]
    out_ref[...] = y * g_ref[...]


def _mlp(gath, w1, tail, W2, b2, W3, b3, gamma):
    return pl.pallas_call(
        _mlp_body,
        grid=(BATCH // RBLK,),
        in_specs=[
            pl.BlockSpec((RBLK, H1), lambda i: (i, 0)),
            pl.BlockSpec((RBLK, 1), lambda i: (i, 0)),
            pl.BlockSpec((1, H1), lambda i: (0, 0)),
            pl.BlockSpec((H2, H1), lambda i: (0, 0)),
            pl.BlockSpec((1, H2), lambda i: (0, 0)),
            pl.BlockSpec((2, H2), lambda i: (0, 0)),
            pl.BlockSpec((1, 2), lambda i: (0, 0)),
            pl.BlockSpec((1, 1), lambda i: (0, 0)),
        ],
        out_specs=pl.BlockSpec((RBLK, 2), lambda i: (i, 0)),
        out_shape=jax.ShapeDtypeStruct((BATCH, 2), jnp.float32),
    )(gath, w1, tail, W2, b2, W3, b3, gamma)


def kernel(indices, weights, offsets, table, W2, b2, W3, b3, gamma):
    del offsets
    indices = indices.astype(jnp.int32)
    hist = _sc_hist(indices, weights).reshape(NC, GPAD)
    gath = _sc_gather(indices, table)
    tail = _tail_matvec(hist, table)
    w1 = weights[:BATCH].reshape(BATCH, 1)
    out = _mlp(gath, w1, tail, W2, b2.reshape(1, H2), W3, b3.reshape(1, 2),
               jnp.reshape(gamma, (1, 1)))
    return out


# speedup vs baseline: 1.0606x; 1.0606x over previous
import functools

import jax
import jax.numpy as jnp
from jax import lax
from jax.experimental import pallas as pl
from jax.experimental.pallas import tpu as pltpu
from jax.experimental.pallas import tpu_sc as plsc

GENOME = 100000
H1 = 128
H2 = 512
BATCH = 16384
NIDX = 327680

NC = 2
NS = 16
NW = NC * NS

ROWS_PER_TILE = BATCH // NW
GROWS = 128
TAIL0 = BATCH
TAIL_N = NIDX - TAIL0
PAIRS_PER_TILE = TAIL_N // NW
PCHUNK = 2432
HROWS = 784
GPAD = HROWS * H1
HCHUNK = 112
GCH = 25088

_SC_MESH = plsc.VectorSubcoreMesh(core_axis_name="c", subcore_axis_name="s")
_SC_PARAMS = pltpu.CompilerParams(needs_layout_passes=False)


def _hist_body(idx_hbm, w_hbm, hist_hbm, hist_v, pi_v, pw_v):
    wid = lax.axis_index("s") * NC + lax.axis_index("c")
    zero16 = jnp.zeros((16,), jnp.float32)
    lane = lax.iota(jnp.int32, 16)

    def _zero(i, carry):
        for u in range(8):
            hist_v[pl.ds(i * 128 + u * 16, 16)] = zero16
        return carry

    lax.fori_loop(0, GPAD // 128, _zero, 0)

    pbase = TAIL0 + wid * PAIRS_PER_TILE
    for c in range(PAIRS_PER_TILE // PCHUNK):
        pltpu.sync_copy(idx_hbm.at[pl.ds(pbase + c * PCHUNK, PCHUNK)], pi_v)
        pltpu.sync_copy(w_hbm.at[pl.ds(pbase + c * PCHUNK, PCHUNK)], pw_v)

        def _scat(v, carry):
            ii = pi_v[pl.ds(v * 16, 16)]
            ww = pw_v[pl.ds(v * 16, 16)]
            ks, ws = plsc.sort_key_val(ii, ww)
            s = plsc.cumsum(ws)
            cnt, last = plsc.scan_count(ks)
            first = jnp.logical_and(cnt == 1, lane > 0)
            plsc.addupdate_scatter(hist_v, [ks], s, mask=last)
            plsc.addupdate_scatter(hist_v, [ks], ws - s, mask=first)
            return carry

        lax.fori_loop(0, PCHUNK // 16, _scat, 0)

    pltpu.sync_copy(hist_v, hist_hbm.at[wid])


_sc_hist = functools.partial(
    pl.kernel,
    out_type=jax.ShapeDtypeStruct((NW, GPAD), jnp.float32),
    mesh=_SC_MESH,
    compiler_params=_SC_PARAMS,
    scratch_types=[
        pltpu.VMEM((GPAD,), jnp.float32),
        pltpu.VMEM((PCHUNK,), jnp.int32),
        pltpu.VMEM((PCHUNK,), jnp.float32),
    ],
)(_hist_body)


def _gath_body(idx_hbm, tab_hbm, gath_hbm, idx_v, rows0_v, rows1_v, sem0, sem1):
    wid = lax.axis_index("s") * NC + lax.axis_index("c")
    base = wid * ROWS_PER_TILE
    pltpu.sync_copy(idx_hbm.at[pl.ds(base, ROWS_PER_TILE)], idx_v)
    bufs = (rows0_v, rows1_v)
    sems = (sem0, sem1)
    nch = ROWS_PER_TILE // GROWS
    copies = []
    for c in range(nch):
        b = c % 2
        copies.append(pltpu.async_copy(
            tab_hbm.at[idx_v.at[pl.ds(c * GROWS, GROWS)]], bufs[b], sems[b]))
        if c >= 1:
            copies[c - 1].wait()
            pltpu.sync_copy(bufs[1 - b],
                            gath_hbm.at[pl.ds(base + (c - 1) * GROWS, GROWS)])
    copies[nch - 1].wait()
    pltpu.sync_copy(bufs[(nch - 1) % 2],
                    gath_hbm.at[pl.ds(base + (nch - 1) * GROWS, GROWS)])


_sc_gather = functools.partial(
    pl.kernel,
    out_type=jax.ShapeDtypeStruct((BATCH, H1), jnp.float32),
    mesh=_SC_MESH,
    compiler_params=_SC_PARAMS,
    scratch_types=[
        pltpu.VMEM((ROWS_PER_TILE,), jnp.int32),
        pltpu.VMEM((GROWS, H1), jnp.float32),
        pltpu.VMEM((GROWS, H1), jnp.float32),
        pltpu.SemaphoreType.DMA,
        pltpu.SemaphoreType.DMA,
    ],
)(_gath_body)


def _leaky(v):
    return jnp.where(v >= 0, v, 0.01 * v)


def _mv_body(acc_ref, tab_ref, out_ref):
    j = pl.program_id(0)
    acc = jnp.sum(acc_ref[...], axis=0, keepdims=True)
    lids = j * GCH + lax.broadcasted_iota(jnp.int32, (1, GCH), 1)
    acc = jnp.where(lids < GENOME, acc, 0.0)
    rids = j * GCH + lax.broadcasted_iota(jnp.int32, (GCH, 1), 0)
    tab = jnp.where(rids < GENOME, tab_ref[...], 0.0)
    part = lax.dot_general(acc, tab, (((1,), (0,)), ((), ())),
                           precision=lax.Precision.HIGHEST,
                           preferred_element_type=jnp.float32)

    @pl.when(j == 0)
    def _():
        out_ref[...] = jnp.zeros_like(out_ref)

    out_ref[...] += part


def _tail_matvec(hist, table):
    return pl.pallas_call(
        _mv_body,
        grid=(GPAD // GCH,),
        in_specs=[
            pl.BlockSpec((NW, GCH), lambda j: (0, j)),
            pl.BlockSpec((GCH, H1), lambda j: (j, 0)),
        ],
        out_specs=pl.BlockSpec((1, H1), lambda j: (0, 0)),
        out_shape=jax.ShapeDtypeStruct((1, H1), jnp.float32),
    )(hist, table)


RBLK = 2048


def _mlp_body(gath_ref, w_ref, tail_ref, W2_ref, b2_ref, W3_ref, b3_ref,
              g_ref, out_ref):
    i = pl.program_id(0)
    x = gath_ref[...] * w_ref[...]
    rid = i * RBLK + lax.broadcasted_iota(jnp.int32, (RBLK, 1), 0)
    is_last = jnp.where(rid == BATCH - 1, 1.0, 0.0)
    x = x + is_last * tail_ref[...]
    x = _leaky(x)
    h = lax.dot_general(x, W2_ref[...], (((1,), (1,)), ((), ())),
                        preferred_element_type=jnp.float32) + b2_ref[...]
    h = _leaky(h)
    y = lax.dot_general(h, W3_ref[...], (((1,), (1,)), ((), ())),
                        preferred_element_type=jnp.float32) + b3_ref[...]
    out_ref[...] = y * g_ref[...]


def _mlp(gath, w1, tail, W2, b2, W3, b3, gamma):
    return pl.pallas_call(
        _mlp_body,
        grid=(BATCH // RBLK,),
        in_specs=[
            pl.BlockSpec((RBLK, H1), lambda i: (i, 0)),
            pl.BlockSpec((RBLK, 1), lambda i: (i, 0)),
            pl.BlockSpec((1, H1), lambda i: (0, 0)),
            pl.BlockSpec((H2, H1), lambda i: (0, 0)),
            pl.BlockSpec((1, H2), lambda i: (0, 0)),
            pl.BlockSpec((2, H2), lambda i: (0, 0)),
            pl.BlockSpec((1, 2), lambda i: (0, 0)),
            pl.BlockSpec((1, 1), lambda i: (0, 0)),
        ],
        out_specs=pl.BlockSpec((RBLK, 2), lambda i: (i, 0)),
        out_shape=jax.ShapeDtypeStruct((BATCH, 2), jnp.float32),
    )(gath, w1, tail, W2, b2, W3, b3, gamma)


def kernel(indices, weights, offsets, table, W2, b2, W3, b3, gamma):
    del offsets
    indices = indices.astype(jnp.int32)
    hist = _sc_hist(indices, weights)
    tail = _tail_matvec(hist, table)
    gath = _sc_gather(indices, table)
    w1 = weights[:BATCH].reshape(BATCH, 1)
    out = _mlp(gath, w1, tail, W2, b2.reshape(1, H2), W3, b3.reshape(1, 2),
               jnp.reshape(gamma, (1, 1)))
    return out


# speedup vs baseline: 1.1280x; 1.0635x over previous
import functools

import jax
import jax.numpy as jnp
from jax import lax
from jax.experimental import pallas as pl
from jax.experimental.pallas import tpu as pltpu
from jax.experimental.pallas import tpu_sc as plsc

GENOME = 100000
H1 = 128
H2 = 512
BATCH = 16384
NIDX = 327680

NC = 2
NS = 16
NW = NC * NS

ROWS_PER_TILE = BATCH // NW
GROWS = 64
NCH = ROWS_PER_TILE // GROWS
TAIL0 = BATCH
TAIL_N = NIDX - TAIL0
PAIRS_PER_TILE = TAIL_N // NW
PCHUNK = 1216
NPC = PAIRS_PER_TILE // PCHUNK
GPAD = 100352
GCH = 25088

_SC_MESH = plsc.VectorSubcoreMesh(core_axis_name="c", subcore_axis_name="s")
_SC_PARAMS = pltpu.CompilerParams(needs_layout_passes=False)


def _sc_body(idx_hbm, w_hbm, tab_hbm, gath_hbm, hist_hbm,
             idxh_v, g0, g1, hist_v, pi0, pw0, pi1, pw1,
             gs0, gs1, ws0, ws1, ps0, ps1):
    wid = lax.axis_index("s") * NC + lax.axis_index("c")
    base = wid * ROWS_PER_TILE
    pbase = TAIL0 + wid * PAIRS_PER_TILE
    gbufs, gsems, wsems = (g0, g1), (gs0, gs1), (ws0, ws1)
    pib, pwb, psems = (pi0, pi1), (pw0, pw1), (ps0, ps1)

    pltpu.sync_copy(idx_hbm.at[pl.ds(base, ROWS_PER_TILE)], idxh_v)
    gcp = [None] * NCH
    wcp = [None] * NCH
    pcp = [None] * NPC
    gcp[0] = pltpu.async_copy(
        tab_hbm.at[idxh_v.at[pl.ds(0, GROWS)]], g0, gs0)
    pcp[0] = (pltpu.async_copy(idx_hbm.at[pl.ds(pbase, PCHUNK)], pi0, ps0),
              pltpu.async_copy(w_hbm.at[pl.ds(pbase, PCHUNK)], pw0, ps0))

    zero16 = jnp.zeros((16,), jnp.float32)

    def _zero(i, carry):
        for u in range(8):
            hist_v[pl.ds(i * 128 + u * 16, 16)] = zero16
        return carry

    lax.fori_loop(0, GPAD // 128, _zero, 0)

    lane = lax.iota(jnp.int32, 16)
    for c in range(NCH):
        b = c % 2
        nb = (c + 1) % 2
        if c >= 1:
            wcp[c - 1].wait()
        if c + 1 < NCH:
            gcp[c + 1] = pltpu.async_copy(
                tab_hbm.at[idxh_v.at[pl.ds((c + 1) * GROWS, GROWS)]],
                gbufs[nb], gsems[nb])
            p0 = pbase + (c + 1) * PCHUNK
            pcp[c + 1] = (
                pltpu.async_copy(idx_hbm.at[pl.ds(p0, PCHUNK)], pib[nb], psems[nb]),
                pltpu.async_copy(w_hbm.at[pl.ds(p0, PCHUNK)], pwb[nb], psems[nb]))
        gcp[c].wait()
        wcp[c] = pltpu.async_copy(
            gbufs[b], gath_hbm.at[pl.ds(base + c * GROWS, GROWS)], wsems[b])
        pcp[c][0].wait()
        pcp[c][1].wait()
        pi_v, pw_v = pib[b], pwb[b]

        def _scat(v, carry):
            ii = pi_v[pl.ds(v * 16, 16)]
            ww = pw_v[pl.ds(v * 16, 16)]
            ks, ws = plsc.sort_key_val(ii, ww)
            s = plsc.cumsum(ws)
            cnt, last = plsc.scan_count(ks)
            first = jnp.logical_and(cnt == 1, lane > 0)
            plsc.addupdate_scatter(hist_v, [ks], s, mask=last)
            plsc.addupdate_scatter(hist_v, [ks], ws - s, mask=first)
            return carry

        lax.fori_loop(0, PCHUNK // 16, _scat, 0)

    wcp[NCH - 1].wait()
    pltpu.sync_copy(hist_v, hist_hbm.at[wid])


_sc_embed = functools.partial(
    pl.kernel,
    out_type=[
        jax.ShapeDtypeStruct((BATCH, H1), jnp.float32),
        jax.ShapeDtypeStruct((NW, GPAD), jnp.float32),
    ],
    mesh=_SC_MESH,
    compiler_params=_SC_PARAMS,
    scratch_types=[
        pltpu.VMEM((ROWS_PER_TILE,), jnp.int32),
        pltpu.VMEM((GROWS, H1), jnp.float32),
        pltpu.VMEM((GROWS, H1), jnp.float32),
        pltpu.VMEM((GPAD,), jnp.float32),
        pltpu.VMEM((PCHUNK,), jnp.int32),
        pltpu.VMEM((PCHUNK,), jnp.float32),
        pltpu.VMEM((PCHUNK,), jnp.int32),
        pltpu.VMEM((PCHUNK,), jnp.float32),
        pltpu.SemaphoreType.DMA,
        pltpu.SemaphoreType.DMA,
        pltpu.SemaphoreType.DMA,
        pltpu.SemaphoreType.DMA,
        pltpu.SemaphoreType.DMA,
        pltpu.SemaphoreType.DMA,
    ],
)(_sc_body)


def _leaky(v):
    return jnp.where(v >= 0, v, 0.01 * v)


def _mv_body(acc_ref, tab_ref, out_ref):
    j = pl.program_id(0)
    acc = jnp.sum(acc_ref[...], axis=0, keepdims=True)
    lids = j * GCH + lax.broadcasted_iota(jnp.int32, (1, GCH), 1)
    acc = jnp.where(lids < GENOME, acc, 0.0)
    rids = j * GCH + lax.broadcasted_iota(jnp.int32, (GCH, 1), 0)
    tab = jnp.where(rids < GENOME, tab_ref[...], 0.0)
    part = lax.dot_general(acc, tab, (((1,), (0,)), ((), ())),
                           precision=lax.Precision.HIGHEST,
                           preferred_element_type=jnp.float32)

    @pl.when(j == 0)
    def _():
        out_ref[...] = jnp.zeros_like(out_ref)

    out_ref[...] += part


def _tail_matvec(hist, table):
    return pl.pallas_call(
        _mv_body,
        grid=(GPAD // GCH,),
        in_specs=[
            pl.BlockSpec((NW, GCH), lambda j: (0, j)),
            pl.BlockSpec((GCH, H1), lambda j: (j, 0)),
        ],
        out_specs=pl.BlockSpec((1, H1), lambda j: (0, 0)),
        out_shape=jax.ShapeDtypeStruct((1, H1), jnp.float32),
    )(hist, table)


RBLK = 2048


def _mlp_body(gath_ref, w_ref, tail_ref, W2_ref, b2_ref, W3_ref, b3_ref,
              g_ref, out_ref):
    i = pl.program_id(0)
    x = gath_ref[...] * w_ref[...]
    rid = i * RBLK + lax.broadcasted_iota(jnp.int32, (RBLK, 1), 0)
    is_last = jnp.where(rid == BATCH - 1, 1.0, 0.0)
    x = x + is_last * tail_ref[...]
    x = _leaky(x)
    h = lax.dot_general(x, W2_ref[...], (((1,), (1,)), ((), ())),
                        preferred_element_type=jnp.float32) + b2_ref[...]
    h = _leaky(h)
    y = lax.dot_general(h, W3_ref[...], (((1,), (1,)), ((), ())),
                        preferred_element_type=jnp.float32) + b3_ref[...]
    out_ref[...] = y * g_ref[...]


def _mlp(gath, w1, tail, W2, b2, W3, b3, gamma):
    return pl.pallas_call(
        _mlp_body,
        grid=(BATCH // RBLK,),
        in_specs=[
            pl.BlockSpec((RBLK, H1), lambda i: (i, 0)),
            pl.BlockSpec((RBLK, 1), lambda i: (i, 0)),
            pl.BlockSpec((1, H1), lambda i: (0, 0)),
            pl.BlockSpec((H2, H1), lambda i: (0, 0)),
            pl.BlockSpec((1, H2), lambda i: (0, 0)),
            pl.BlockSpec((2, H2), lambda i: (0, 0)),
            pl.BlockSpec((1, 2), lambda i: (0, 0)),
            pl.BlockSpec((1, 1), lambda i: (0, 0)),
        ],
        out_specs=pl.BlockSpec((RBLK, 2), lambda i: (i, 0)),
        out_shape=jax.ShapeDtypeStruct((BATCH, 2), jnp.float32),
    )(gath, w1, tail, W2, b2, W3, b3, gamma)


def kernel(indices, weights, offsets, table, W2, b2, W3, b3, gamma):
    del offsets
    indices = indices.astype(jnp.int32)
    gath, hist = _sc_embed(indices, weights, table)
    tail = _tail_matvec(hist, table)
    w1 = weights[:BATCH].reshape(BATCH, 1)
    out = _mlp(gath, w1, tail, W2, b2.reshape(1, H2), W3, b3.reshape(1, 2),
               jnp.reshape(gamma, (1, 1)))
    return out
